# Initial kernel scaffold; baseline (speedup 1.0000x reference)
import jax
import jax.numpy as jnp
from jax.experimental import pallas as pl


def kernel(x, in_w, in_b, out_w, out_b, codebooks):
    raise NotImplementedError("write your pallas kernel here")


# speedup vs baseline: 1.3445x; 1.3445x over previous
import functools

import jax
import jax.numpy as jnp
from jax.experimental import pallas as pl
from jax.experimental.pallas import tpu as pltpu

_COMMIT = 0.25


def _body(nq, nc, chunk, xt_ref, win_ref, bin_ref, wout_ref, bout_ref, cb_ref,
          qout_ref, idx_ref, loss_ref, r_ref, acc_ref):
    q = pl.program_id(0)
    c = pl.program_id(1)

    @pl.when((q == 0) & (c == 0))
    def _init():
        r_ref[...] = xt_ref[...]

    sl = pl.ds(c * chunk, chunk)
    rc = r_ref[sl, :]
    w_in = win_ref[0]
    z = jax.lax.dot_general(rc, w_in, (((1,), (1,)), ((), ())),
                            preferred_element_type=jnp.float32)
    z = z + bin_ref[0]

    zn = jnp.sqrt(jnp.sum(z * z, axis=1, keepdims=True))
    en = z / (zn + 1e-12)

    cb = cb_ref[0]
    cbn = jnp.sqrt(jnp.sum(cb * cb, axis=1, keepdims=True))
    cn = cb / (cbn + 1e-12)

    s = jax.lax.dot_general(en, cn, (((1,), (1,)), ((), ())),
                            preferred_element_type=jnp.float32)
    en2 = jnp.sum(en * en, axis=1, keepdims=True)
    cn2 = jnp.sum(cn * cn, axis=1, keepdims=True)
    cn2_row = jnp.reshape(cn2, (1, cn2.shape[0]))
    neg_dist = -((en2 - 2.0 * s) + cn2_row)

    kk = neg_dist.shape[1]
    lane = jax.lax.broadcasted_iota(jnp.int32, neg_dist.shape, 1)
    mx = jnp.max(neg_dist, axis=1, keepdims=True)
    cand = jnp.where(neg_dist == mx, lane, kk)
    idx = jnp.min(cand, axis=1, keepdims=True)
    idx_ref[0, 0] = idx

    oh = (lane == idx).astype(jnp.float32)
    zq = jax.lax.dot_general(oh, cb, (((1,), (0,)), ((), ())),
                             preferred_element_type=jnp.float32,
                             precision=jax.lax.Precision.HIGHEST)

    diff = z - zq
    part = jnp.sum(diff * diff)

    @pl.when(c == 0)
    def _acc0():
        acc_ref[0] = part

    @pl.when(c != 0)
    def _accn():
        acc_ref[0] = acc_ref[0] + part

    n_elems = xt_ref.shape[0] * z.shape[1]

    @pl.when(c == nc - 1)
    def _loss():
        m = acc_ref[0] / jnp.float32(n_elems)
        loss_ref[0, 0, 0] = m * _COMMIT + m

    zq_st = z + (zq - z)
    w_out = wout_ref[0]
    out = jax.lax.dot_general(zq_st, w_out, (((1,), (1,)), ((), ())),
                              preferred_element_type=jnp.float32)
    out = out + bout_ref[0]

    r_new = rc - out
    r_ref[sl, :] = r_new

    @pl.when(q == nq - 1)
    def _write_q():
        qout_ref[...] = xt_ref[sl, :] - r_new


def kernel(x, in_w, in_b, out_w, out_b, codebooks):
    b, d, t = x.shape
    nq, cd, _ = in_w.shape
    k = codebooks.shape[1]
    bt = b * t
    chunk = 512
    nc = bt // chunk

    xt = jnp.transpose(x, (0, 2, 1)).reshape(bt, d)
    inb3 = in_b.reshape(nq, 1, cd)
    outb3 = out_b.reshape(nq, 1, d)

    grid = (nq, nc)
    body = functools.partial(_body, nq, nc, chunk)

    qout, idx4, loss2 = pl.pallas_call(
        body,
        grid=grid,
        in_specs=[
            pl.BlockSpec((bt, d), lambda q, c: (0, 0)),
            pl.BlockSpec((1, cd, d), lambda q, c: (q, 0, 0)),
            pl.BlockSpec((1, 1, cd), lambda q, c: (q, 0, 0)),
            pl.BlockSpec((1, d, cd), lambda q, c: (q, 0, 0)),
            pl.BlockSpec((1, 1, d), lambda q, c: (q, 0, 0)),
            pl.BlockSpec((1, k, cd), lambda q, c: (q, 0, 0)),
        ],
        out_specs=[
            pl.BlockSpec((chunk, d), lambda q, c: (jnp.where(q == nq - 1, c, 0), 0)),
            pl.BlockSpec((1, 1, chunk, 1), lambda q, c: (q, c, 0, 0)),
            pl.BlockSpec(memory_space=pltpu.SMEM, block_shape=(1, 1, 1),
                         index_map=lambda q, c: (q, 0, 0)),
        ],
        out_shape=[
            jax.ShapeDtypeStruct((bt, d), jnp.float32),
            jax.ShapeDtypeStruct((nq, nc, chunk, 1), jnp.int32),
            jax.ShapeDtypeStruct((nq, 1, 1), jnp.float32),
        ],
        scratch_shapes=[
            pltpu.VMEM((bt, d), jnp.float32),
            pltpu.SMEM((1,), jnp.float32),
        ],
        compiler_params=pltpu.CompilerParams(
            dimension_semantics=("arbitrary", "arbitrary"),
        ),
    )(xt, in_w, inb3, out_w, outb3, codebooks)

    quantized = jnp.transpose(qout.reshape(b, t, d), (0, 2, 1))
    indices = idx4.reshape(nq, b, t)
    losses = loss2.reshape(nq)
    return quantized, indices, losses


# speedup vs baseline: 2.0007x; 1.4881x over previous
import functools

import jax
import jax.numpy as jnp
from jax.experimental import pallas as pl
from jax.experimental.pallas import tpu as pltpu

_COMMIT = 0.25


def _body(nq, nc, chunk, xt_ref, win_ref, bin_ref, wout_ref, bout_ref, cb_ref,
          qout_ref, idx_ref, loss_ref, r_ref, acc_ref, cn_ref, cn2_ref):
    q = pl.program_id(0)
    c = pl.program_id(1)

    @pl.when((q == 0) & (c == 0))
    def _init():
        r_ref[...] = xt_ref[...]

    @pl.when(c == 0)
    def _prep_cb():
        cbf = cb_ref[0]
        cbn = jnp.sqrt(jnp.sum(cbf * cbf, axis=1, keepdims=True))
        cnf = cbf / (cbn + 1e-12)
        cn_ref[...] = cnf
        cn2 = jnp.sum(cnf * cnf, axis=1, keepdims=True)
        cn2_ref[...] = jnp.reshape(cn2, (1, cn2.shape[0]))

    sl = pl.ds(c * chunk, chunk)
    rc = r_ref[sl, :]
    w_in = win_ref[0]
    z = jax.lax.dot_general(rc, w_in, (((1,), (1,)), ((), ())),
                            preferred_element_type=jnp.float32)
    z = z + bin_ref[0]

    zn = jnp.sqrt(jnp.sum(z * z, axis=1, keepdims=True))
    en = z / (zn + 1e-12)

    s = jax.lax.dot_general(en, cn_ref[...], (((1,), (1,)), ((), ())),
                            preferred_element_type=jnp.float32)
    en2 = jnp.sum(en * en, axis=1, keepdims=True)
    dist = (en2 - 2.0 * s) + cn2_ref[...]

    kk = dist.shape[1]
    lane = jax.lax.broadcasted_iota(jnp.int32, dist.shape, 1)
    mn = jnp.min(dist, axis=1, keepdims=True)
    cand = jnp.where(dist == mn, lane, kk)
    idx = jnp.min(cand, axis=1, keepdims=True)
    idx_ref[0, 0] = idx

    oh = (lane == idx).astype(jnp.float32)
    zq = jax.lax.dot_general(oh, cb_ref[0], (((1,), (0,)), ((), ())),
                             preferred_element_type=jnp.float32)

    diff = z - zq
    part = jnp.sum(diff * diff)

    @pl.when(c == 0)
    def _acc0():
        acc_ref[0] = part

    @pl.when(c != 0)
    def _accn():
        acc_ref[0] = acc_ref[0] + part

    n_elems = xt_ref.shape[0] * z.shape[1]

    @pl.when(c == nc - 1)
    def _loss():
        m = acc_ref[0] / jnp.float32(n_elems)
        loss_ref[0, 0, 0] = m * _COMMIT + m

    zq_st = z + (zq - z)
    w_out = wout_ref[0]
    out = jax.lax.dot_general(zq_st, w_out, (((1,), (1,)), ((), ())),
                              preferred_element_type=jnp.float32)
    out = out + bout_ref[0]

    r_new = rc - out
    r_ref[sl, :] = r_new

    @pl.when(q == nq - 1)
    def _write_q():
        qout_ref[...] = xt_ref[sl, :] - r_new


def kernel(x, in_w, in_b, out_w, out_b, codebooks):
    b, d, t = x.shape
    nq, cd, _ = in_w.shape
    k = codebooks.shape[1]
    bt = b * t
    chunk = 512
    nc = bt // chunk

    xt = jnp.transpose(x, (0, 2, 1)).reshape(bt, d)
    inb3 = in_b.reshape(nq, 1, cd)
    outb3 = out_b.reshape(nq, 1, d)

    grid = (nq, nc)
    body = functools.partial(_body, nq, nc, chunk)

    qout, idx4, loss2 = pl.pallas_call(
        body,
        grid=grid,
        in_specs=[
            pl.BlockSpec((bt, d), lambda q, c: (0, 0)),
            pl.BlockSpec((1, cd, d), lambda q, c: (q, 0, 0)),
            pl.BlockSpec((1, 1, cd), lambda q, c: (q, 0, 0)),
            pl.BlockSpec((1, d, cd), lambda q, c: (q, 0, 0)),
            pl.BlockSpec((1, 1, d), lambda q, c: (q, 0, 0)),
            pl.BlockSpec((1, k, cd), lambda q, c: (q, 0, 0)),
        ],
        out_specs=[
            pl.BlockSpec((chunk, d), lambda q, c: (jnp.where(q == nq - 1, c, 0), 0)),
            pl.BlockSpec((1, 1, chunk, 1), lambda q, c: (q, c, 0, 0)),
            pl.BlockSpec(memory_space=pltpu.SMEM, block_shape=(1, 1, 1),
                         index_map=lambda q, c: (q, 0, 0)),
        ],
        out_shape=[
            jax.ShapeDtypeStruct((bt, d), jnp.float32),
            jax.ShapeDtypeStruct((nq, nc, chunk, 1), jnp.int32),
            jax.ShapeDtypeStruct((nq, 1, 1), jnp.float32),
        ],
        scratch_shapes=[
            pltpu.VMEM((bt, d), jnp.float32),
            pltpu.SMEM((1,), jnp.float32),
            pltpu.VMEM((k, cd), jnp.float32),
            pltpu.VMEM((1, k), jnp.float32),
        ],
        compiler_params=pltpu.CompilerParams(
            dimension_semantics=("arbitrary", "arbitrary"),
        ),
    )(xt, in_w, inb3, out_w, outb3, codebooks)

    quantized = jnp.transpose(qout.reshape(b, t, d), (0, 2, 1))
    indices = idx4.reshape(nq, b, t)
    losses = loss2.reshape(nq)
    return quantized, indices, losses
